# baseline (speedup 1.0000x reference)
import functools

import jax
import jax.numpy as jnp
from jax import lax
from jax.experimental import pallas as pl
from jax.experimental.pallas import tpu as pltpu
from jax.experimental.pallas import tpu_sc as plsc

TM = 1024
NW = 32
TPW = 256
CHUNK = 32
CCHUNK = 16


def _route_body(x_ref, wg_ref, bg_ref, pos_ref, te_ref, logits_s):
    i = pl.program_id(0)
    nt = pl.num_programs(0)
    tm = x_ref.shape[0]
    logits_s[pl.ds(i * tm, tm), :] = (
        lax.dot_general(
            x_ref[...], wg_ref[...], (((1,), (0,)), ((), ())),
            preferred_element_type=jnp.float32,
            precision=lax.Precision.DEFAULT,
        )
        + bg_ref[...]
    )

    @pl.when(i == nt - 1)
    def _route():
        logits = logits_s[...]
        t, e = logits.shape
        nw = t // TPW
        ecol = lax.broadcasted_iota(jnp.int32, (t, e), 1)
        m1 = jnp.max(logits, axis=1, keepdims=True)
        idx1 = jnp.min(jnp.where(logits == m1, ecol, e), axis=1, keepdims=True)
        mask1 = ecol == idx1
        l2 = jnp.where(mask1, -jnp.inf, logits)
        m2 = jnp.max(l2, axis=1, keepdims=True)
        idx2 = jnp.min(jnp.where(l2 == m2, ecol, e), axis=1, keepdims=True)
        mask2 = ecol == idx2

        oh = jnp.concatenate(
            [mask1.astype(jnp.int32), mask2.astype(jnp.int32)], axis=1)
        rowmod = lax.broadcasted_iota(jnp.int32, (t, 1), 0) % TPW
        c = oh
        s = 1
        while s < TPW:
            sh = jnp.concatenate(
                [jnp.zeros((s, 2 * e), jnp.int32), c[: t - s, :]], axis=0)
            c = c + jnp.where(rowmod >= s, sh, 0)
            s *= 2
        ex = c - oh

        oh8 = mask1.astype(jnp.int32) + mask2.astype(jnp.int32)
        half = t // 2
        h0 = jnp.sum(oh8[:half], axis=0, keepdims=True)
        h1 = jnp.sum(oh8[half:], axis=0, keepdims=True)

        starts = []
        h0s = []
        run = jnp.int32(0)
        for j in range(e):
            starts.append(run)
            h0s.append(h0[0, j])
            tot = h0[0, j] + h1[0, j]
            run = ((run + tot + TM - 1) // TM) * TM

        selr = (lax.broadcasted_iota(jnp.int32, (nw, t), 1)
                - lax.broadcasted_iota(jnp.int32, (nw, t), 0) * TPW) == TPW - 1
        cw = lax.dot_general(
            selr.astype(jnp.float32), c.astype(jnp.float32),
            (((1,), (0,)), ((), ())),
            preferred_element_type=jnp.float32,
            precision=lax.Precision.HIGHEST,
        ).astype(jnp.int32)
        cnt1w = cw[:, :e]
        tot8 = cnt1w + cw[:, e:]
        rw = lax.broadcasted_iota(jnp.int32, (nw, 1), 0) % (nw // 2)
        ac = tot8
        ss = 1
        while ss < nw // 2:
            shw = jnp.concatenate(
                [jnp.zeros((ss, e), jnp.int32), ac[: nw - ss, :]], axis=0)
            ac = ac + jnp.where(rw >= ss, shw, 0)
            ss *= 2
        ex8 = ac - tot8
        lane8 = lax.broadcasted_iota(jnp.int32, (nw, e), 1)
        row8 = lax.broadcasted_iota(jnp.int32, (nw, e), 0)
        st8 = jnp.zeros((nw, e), jnp.int32)
        for j in range(e):
            st8 = st8 + jnp.where(lane8 == j, starts[j], 0)
            st8 = st8 + jnp.where((lane8 == j) & (row8 >= nw // 2), h0s[j], 0)
        b1w = ex8 + st8
        b2w = b1w + cnt1w
        bb = jnp.concatenate([b1w, b2w], axis=1)

        bsel = (lax.broadcasted_iota(jnp.int32, (t, nw), 0) // TPW
                == lax.broadcasted_iota(jnp.int32, (t, nw), 1))
        badd = lax.dot_general(
            bsel.astype(jnp.float32), bb.astype(jnp.float32),
            (((1,), (0,)), ((), ())),
            preferred_element_type=jnp.float32,
            precision=lax.Precision.HIGHEST,
        ).astype(jnp.int32)

        posb = oh * (ex + badd)
        lane2e = lax.broadcasted_iota(jnp.int32, (1, 2 * e), 1)
        pos1 = jnp.sum(jnp.where(lane2e < e, posb, 0), axis=1, keepdims=True)
        pos2 = jnp.sum(jnp.where(lane2e >= e, posb, 0), axis=1, keepdims=True)
        pos_ref[...] = jnp.where(ecol == 0, pos1, 0) + jnp.where(
            ecol == 1, pos2, 0)

        tcol = lax.broadcasted_iota(jnp.int32, te_ref.shape, 1)
        te = jnp.zeros(te_ref.shape, jnp.int32)
        for j in range(1, e):
            te = te + jnp.where(tcol * TM >= starts[j], 1, 0)
        te_ref[...] = te


def _dispatch_body(xt_hbm, p1_hbm, p2_hbm, xs_hbm, i1_v, i2_v, rows_v,
                   sem_in, sem_o0, sem_o1):
    c = lax.axis_index("c")
    s = lax.axis_index("s")
    w = c * 16 + s
    pltpu.sync_copy(p1_hbm.at[w], i1_v)
    pltpu.sync_copy(p2_hbm.at[w], i2_v)
    nch = TPW // CHUNK
    base = w * TPW
    osems = [sem_o0, sem_o1]
    ins = [pltpu.async_copy(xt_hbm.at[pl.ds(base, CHUNK)], rows_v.at[0],
                            sem_in)]
    pend = [None, None]
    for g in range(nch):
        b = g % 2
        ins[g].wait()
        if pend[b] is not None:
            pend[b][0].wait()
            pend[b][1].wait()
        c1 = pltpu.async_copy(rows_v.at[b], xs_hbm.at[i1_v.at[g]], osems[b])
        c2 = pltpu.async_copy(rows_v.at[b], xs_hbm.at[i2_v.at[g]], osems[b])
        pend[b] = (c1, c2)
        if g + 1 < nch:
            nb = (g + 1) % 2
            if pend[nb] is not None:
                pend[nb][0].wait()
                pend[nb][1].wait()
                pend[nb] = None
            ins.append(pltpu.async_copy(
                xt_hbm.at[pl.ds(base + (g + 1) * CHUNK, CHUNK)],
                rows_v.at[nb], sem_in))
    for p in pend:
        if p is not None:
            p[0].wait()
            p[1].wait()


def _gmm_body(te_ref, xs_ref, we_ref, be_ref, o_ref):
    del te_ref
    y = lax.dot_general(
        xs_ref[...], we_ref[0], (((1,), (0,)), ((), ())),
        preferred_element_type=jnp.float32,
        precision=lax.Precision.DEFAULT,
    )
    o_ref[...] = jnp.maximum(y + be_ref[0], 0.0)


def _combine_body(rows_hbm, p1_hbm, p2_hbm, o_hbm, i1_v, i2_v, b1_v, b2_v,
                  sem_e, sem_o):
    c = lax.axis_index("c")
    s = lax.axis_index("s")
    w = c * 16 + s
    pltpu.sync_copy(p1_hbm.at[w], i1_v)
    pltpu.sync_copy(p2_hbm.at[w], i2_v)
    base = w * TPW
    d = b1_v.shape[2]
    nch = TPW // CCHUNK
    sems = [sem_e, sem_o]

    for sub in range(2):
        pltpu.async_copy(rows_hbm.at[i1_v.at[sub]], b1_v.at[sub], sems[sub])
        pltpu.async_copy(rows_hbm.at[i2_v.at[sub]], b2_v.at[sub], sems[sub])

    def gbody(gg, _):
        for sub in range(2):
            g = 2 * gg + sub
            pltpu.make_async_copy(rows_hbm.at[i1_v.at[g]], b1_v.at[sub],
                                  sems[sub]).wait()
            pltpu.make_async_copy(rows_hbm.at[i2_v.at[g]], b2_v.at[sub],
                                  sems[sub]).wait()
            for r in range(CCHUNK):
                @plsc.parallel_loop(0, d, step=16, unroll=8)
                def _avg(cc, r=r, sub=sub):
                    b1_v[sub, r, pl.ds(cc, 16)] = (
                        b1_v[sub, r, pl.ds(cc, 16)]
                        + b2_v[sub, r, pl.ds(cc, 16)]) * 0.5
            pltpu.sync_copy(b1_v.at[sub],
                            o_hbm.at[pl.ds(base + g * CCHUNK, CCHUNK)])

            @pl.when(g + 2 < nch)
            def _next(g=g, sub=sub):
                pltpu.async_copy(rows_hbm.at[i1_v.at[g + 2]], b1_v.at[sub],
                                 sems[sub])
                pltpu.async_copy(rows_hbm.at[i2_v.at[g + 2]], b2_v.at[sub],
                                 sems[sub])
        return 0

    lax.fori_loop(0, nch // 2, gbody, 0)


@functools.partial(jax.jit, static_argnums=())
def kernel(x, Wg, bg, We, be):
    n, s, v = x.shape
    e = Wg.shape[1]
    out = We.shape[2]
    t = n * s
    xt = x.reshape(t, v)
    tm_g = 1024
    nt_g = t // tm_g

    rows_pad = ((2 * t + e * (TM - 1)) // TM + 1) * TM
    n_tiles = rows_pad // TM

    posout, te = pl.pallas_call(
        _route_body,
        grid=(nt_g,),
        in_specs=[
            pl.BlockSpec((tm_g, v), lambda i: (i, 0)),
            pl.BlockSpec((v, e), lambda i: (0, 0)),
            pl.BlockSpec((1, e), lambda i: (0, 0)),
        ],
        out_specs=[
            pl.BlockSpec((t, e), lambda i: (0, 0)),
            pl.BlockSpec((1, 128), lambda i: (0, 0)),
        ],
        out_shape=[
            jax.ShapeDtypeStruct((t, e), jnp.int32),
            jax.ShapeDtypeStruct((1, 128), jnp.int32),
        ],
        scratch_shapes=[pltpu.VMEM((t, e), jnp.float32)],
    )(xt, Wg, bg.reshape(1, e))

    nch = TPW // CHUNK
    pos1 = posout[:, 0].reshape(NW, nch, CHUNK)
    pos2 = posout[:, 1].reshape(NW, nch, CHUNK)
    te_flat = te.reshape(128)[:n_tiles]

    mesh = plsc.VectorSubcoreMesh(core_axis_name="c", subcore_axis_name="s")

    xs = pl.kernel(
        _dispatch_body,
        out_type=jax.ShapeDtypeStruct((rows_pad, v), jnp.float32),
        mesh=mesh,
        scratch_types=[
            pltpu.VMEM((nch, CHUNK), jnp.int32),
            pltpu.VMEM((nch, CHUNK), jnp.int32),
            pltpu.VMEM((2, CHUNK, v), jnp.float32),
            pltpu.SemaphoreType.DMA,
            pltpu.SemaphoreType.DMA,
            pltpu.SemaphoreType.DMA,
        ],
    )(xt, pos1, pos2)

    orows = pl.pallas_call(
        _gmm_body,
        grid_spec=pltpu.PrefetchScalarGridSpec(
            num_scalar_prefetch=1,
            grid=(n_tiles,),
            in_specs=[
                pl.BlockSpec((TM, v), lambda i, te_r: (i, 0)),
                pl.BlockSpec((1, v, out), lambda i, te_r: (te_r[i], 0, 0)),
                pl.BlockSpec((1, 1, out), lambda i, te_r: (te_r[i], 0, 0)),
            ],
            out_specs=pl.BlockSpec((TM, out), lambda i, te_r: (i, 0)),
        ),
        out_shape=jax.ShapeDtypeStruct((rows_pad, out), jnp.float32),
    )(te_flat, xs, We, be.reshape(e, 1, out))

    nchc = TPW // CCHUNK
    p1c = posout[:, 0].reshape(NW, nchc, CCHUNK)
    p2c = posout[:, 1].reshape(NW, nchc, CCHUNK)

    o = pl.kernel(
        _combine_body,
        out_type=jax.ShapeDtypeStruct((t, out), jnp.float32),
        mesh=mesh,
        scratch_types=[
            pltpu.VMEM((nchc, CCHUNK), jnp.int32),
            pltpu.VMEM((nchc, CCHUNK), jnp.int32),
            pltpu.VMEM((2, CCHUNK, out), jnp.float32),
            pltpu.VMEM((2, CCHUNK, out), jnp.float32),
            pltpu.SemaphoreType.DMA,
            pltpu.SemaphoreType.DMA,
        ],
    )(orows, p1c, p2c)

    return o.reshape(n, s, out)


# speedup vs baseline: 1.0103x; 1.0103x over previous
import functools

import jax
import jax.numpy as jnp
from jax import lax
from jax.experimental import pallas as pl
from jax.experimental.pallas import tpu as pltpu
from jax.experimental.pallas import tpu_sc as plsc

TM = 512
NW = 32
TPW = 256
CHUNK = 32
CCHUNK = 16


def _route_body(x_ref, wg_ref, bg_ref, pos_ref, te_ref, logits_s):
    i = pl.program_id(0)
    nt = pl.num_programs(0)
    tm = x_ref.shape[0]
    logits_s[pl.ds(i * tm, tm), :] = (
        lax.dot_general(
            x_ref[...], wg_ref[...], (((1,), (0,)), ((), ())),
            preferred_element_type=jnp.float32,
            precision=lax.Precision.DEFAULT,
        )
        + bg_ref[...]
    )

    @pl.when(i == nt - 1)
    def _route():
        logits = logits_s[...]
        t, e = logits.shape
        nw = t // TPW
        ecol = lax.broadcasted_iota(jnp.int32, (t, e), 1)
        m1 = jnp.max(logits, axis=1, keepdims=True)
        idx1 = jnp.min(jnp.where(logits == m1, ecol, e), axis=1, keepdims=True)
        mask1 = ecol == idx1
        l2 = jnp.where(mask1, -jnp.inf, logits)
        m2 = jnp.max(l2, axis=1, keepdims=True)
        idx2 = jnp.min(jnp.where(l2 == m2, ecol, e), axis=1, keepdims=True)
        mask2 = ecol == idx2

        oh = jnp.concatenate(
            [mask1.astype(jnp.int32), mask2.astype(jnp.int32)], axis=1)
        rowmod = lax.broadcasted_iota(jnp.int32, (t, 1), 0) % TPW
        c = oh
        s = 1
        while s < TPW:
            sh = jnp.concatenate(
                [jnp.zeros((s, 2 * e), jnp.int32), c[: t - s, :]], axis=0)
            c = c + jnp.where(rowmod >= s, sh, 0)
            s *= 2
        ex = c - oh

        oh8 = mask1.astype(jnp.int32) + mask2.astype(jnp.int32)
        half = t // 2
        h0 = jnp.sum(oh8[:half], axis=0, keepdims=True)
        h1 = jnp.sum(oh8[half:], axis=0, keepdims=True)

        starts = []
        h0s = []
        run = jnp.int32(0)
        for j in range(e):
            starts.append(run)
            h0s.append(h0[0, j])
            tot = h0[0, j] + h1[0, j]
            run = ((run + tot + TM - 1) // TM) * TM

        selr = (lax.broadcasted_iota(jnp.int32, (nw, t), 1)
                - lax.broadcasted_iota(jnp.int32, (nw, t), 0) * TPW) == TPW - 1
        cw = lax.dot_general(
            selr.astype(jnp.float32), c.astype(jnp.float32),
            (((1,), (0,)), ((), ())),
            preferred_element_type=jnp.float32,
            precision=lax.Precision.HIGHEST,
        ).astype(jnp.int32)
        cnt1w = cw[:, :e]
        tot8 = cnt1w + cw[:, e:]
        rw = lax.broadcasted_iota(jnp.int32, (nw, 1), 0) % (nw // 2)
        ac = tot8
        ss = 1
        while ss < nw // 2:
            shw = jnp.concatenate(
                [jnp.zeros((ss, e), jnp.int32), ac[: nw - ss, :]], axis=0)
            ac = ac + jnp.where(rw >= ss, shw, 0)
            ss *= 2
        ex8 = ac - tot8
        lane8 = lax.broadcasted_iota(jnp.int32, (nw, e), 1)
        row8 = lax.broadcasted_iota(jnp.int32, (nw, e), 0)
        st8 = jnp.zeros((nw, e), jnp.int32)
        for j in range(e):
            st8 = st8 + jnp.where(lane8 == j, starts[j], 0)
            st8 = st8 + jnp.where((lane8 == j) & (row8 >= nw // 2), h0s[j], 0)
        b1w = ex8 + st8
        b2w = b1w + cnt1w
        bb = jnp.concatenate([b1w, b2w], axis=1)

        bsel = (lax.broadcasted_iota(jnp.int32, (t, nw), 0) // TPW
                == lax.broadcasted_iota(jnp.int32, (t, nw), 1))
        badd = lax.dot_general(
            bsel.astype(jnp.float32), bb.astype(jnp.float32),
            (((1,), (0,)), ((), ())),
            preferred_element_type=jnp.float32,
            precision=lax.Precision.HIGHEST,
        ).astype(jnp.int32)

        posb = oh * (ex + badd)
        lane2e = lax.broadcasted_iota(jnp.int32, (1, 2 * e), 1)
        pos1 = jnp.sum(jnp.where(lane2e < e, posb, 0), axis=1, keepdims=True)
        pos2 = jnp.sum(jnp.where(lane2e >= e, posb, 0), axis=1, keepdims=True)
        pos_ref[...] = jnp.where(ecol == 0, pos1, 0) + jnp.where(
            ecol == 1, pos2, 0)

        tcol = lax.broadcasted_iota(jnp.int32, te_ref.shape, 1)
        te = jnp.zeros(te_ref.shape, jnp.int32)
        for j in range(1, e):
            te = te + jnp.where(tcol * TM >= starts[j], 1, 0)
        te_ref[...] = te


def _dispatch_body(xt_hbm, p1_hbm, p2_hbm, xs_hbm, i1_v, i2_v, rows_v,
                   sem_in, sem_o0, sem_o1):
    c = lax.axis_index("c")
    s = lax.axis_index("s")
    w = c * 16 + s
    pltpu.sync_copy(p1_hbm.at[w], i1_v)
    pltpu.sync_copy(p2_hbm.at[w], i2_v)
    nch = TPW // CHUNK
    base = w * TPW
    osems = [sem_o0, sem_o1]
    ins = [pltpu.async_copy(xt_hbm.at[pl.ds(base, CHUNK)], rows_v.at[0],
                            sem_in)]
    pend = [None, None]
    for g in range(nch):
        b = g % 2
        ins[g].wait()
        if pend[b] is not None:
            pend[b][0].wait()
            pend[b][1].wait()
        c1 = pltpu.async_copy(rows_v.at[b], xs_hbm.at[i1_v.at[g]], osems[b])
        c2 = pltpu.async_copy(rows_v.at[b], xs_hbm.at[i2_v.at[g]], osems[b])
        pend[b] = (c1, c2)
        if g + 1 < nch:
            nb = (g + 1) % 2
            if pend[nb] is not None:
                pend[nb][0].wait()
                pend[nb][1].wait()
                pend[nb] = None
            ins.append(pltpu.async_copy(
                xt_hbm.at[pl.ds(base + (g + 1) * CHUNK, CHUNK)],
                rows_v.at[nb], sem_in))
    for p in pend:
        if p is not None:
            p[0].wait()
            p[1].wait()


def _gmm_body(te_ref, xs_ref, we_ref, be_ref, o_ref):
    del te_ref
    y = lax.dot_general(
        xs_ref[...], we_ref[0], (((1,), (0,)), ((), ())),
        preferred_element_type=jnp.float32,
        precision=lax.Precision.DEFAULT,
    )
    o_ref[...] = jnp.maximum(y + be_ref[0], 0.0)


def _combine_body(rows_hbm, p1_hbm, p2_hbm, o_hbm, i1_v, i2_v, b1_v, b2_v,
                  sem_e, sem_o):
    c = lax.axis_index("c")
    s = lax.axis_index("s")
    w = c * 16 + s
    pltpu.sync_copy(p1_hbm.at[w], i1_v)
    pltpu.sync_copy(p2_hbm.at[w], i2_v)
    base = w * TPW
    d = b1_v.shape[2]
    nch = TPW // CCHUNK
    sems = [sem_e, sem_o]

    for sub in range(2):
        pltpu.async_copy(rows_hbm.at[i1_v.at[sub]], b1_v.at[sub], sems[sub])
        pltpu.async_copy(rows_hbm.at[i2_v.at[sub]], b2_v.at[sub], sems[sub])

    def gbody(gg, _):
        for sub in range(2):
            g = 2 * gg + sub
            pltpu.make_async_copy(rows_hbm.at[i1_v.at[g]], b1_v.at[sub],
                                  sems[sub]).wait()
            pltpu.make_async_copy(rows_hbm.at[i2_v.at[g]], b2_v.at[sub],
                                  sems[sub]).wait()
            for r in range(CCHUNK):
                @plsc.parallel_loop(0, d, step=16, unroll=8)
                def _avg(cc, r=r, sub=sub):
                    b1_v[sub, r, pl.ds(cc, 16)] = (
                        b1_v[sub, r, pl.ds(cc, 16)]
                        + b2_v[sub, r, pl.ds(cc, 16)]) * 0.5
            pltpu.sync_copy(b1_v.at[sub],
                            o_hbm.at[pl.ds(base + g * CCHUNK, CCHUNK)])

            @pl.when(g + 2 < nch)
            def _next(g=g, sub=sub):
                pltpu.async_copy(rows_hbm.at[i1_v.at[g + 2]], b1_v.at[sub],
                                 sems[sub])
                pltpu.async_copy(rows_hbm.at[i2_v.at[g + 2]], b2_v.at[sub],
                                 sems[sub])
        return 0

    lax.fori_loop(0, nch // 2, gbody, 0)


@functools.partial(jax.jit, static_argnums=())
def kernel(x, Wg, bg, We, be):
    n, s, v = x.shape
    e = Wg.shape[1]
    out = We.shape[2]
    t = n * s
    xt = x.reshape(t, v)
    tm_g = 1024
    nt_g = t // tm_g

    rows_pad = ((2 * t + e * (TM - 1)) // TM + 1) * TM
    n_tiles = rows_pad // TM

    posout, te = pl.pallas_call(
        _route_body,
        grid=(nt_g,),
        in_specs=[
            pl.BlockSpec((tm_g, v), lambda i: (i, 0)),
            pl.BlockSpec((v, e), lambda i: (0, 0)),
            pl.BlockSpec((1, e), lambda i: (0, 0)),
        ],
        out_specs=[
            pl.BlockSpec((t, e), lambda i: (0, 0)),
            pl.BlockSpec((1, 128), lambda i: (0, 0)),
        ],
        out_shape=[
            jax.ShapeDtypeStruct((t, e), jnp.int32),
            jax.ShapeDtypeStruct((1, 128), jnp.int32),
        ],
        scratch_shapes=[pltpu.VMEM((t, e), jnp.float32)],
    )(xt, Wg, bg.reshape(1, e))

    nch = TPW // CHUNK
    pos1 = posout[:, 0].reshape(NW, nch, CHUNK)
    pos2 = posout[:, 1].reshape(NW, nch, CHUNK)
    te_flat = te.reshape(128)[:n_tiles]

    mesh = plsc.VectorSubcoreMesh(core_axis_name="c", subcore_axis_name="s")

    xs = pl.kernel(
        _dispatch_body,
        out_type=jax.ShapeDtypeStruct((rows_pad, v), jnp.float32),
        mesh=mesh,
        scratch_types=[
            pltpu.VMEM((nch, CHUNK), jnp.int32),
            pltpu.VMEM((nch, CHUNK), jnp.int32),
            pltpu.VMEM((2, CHUNK, v), jnp.float32),
            pltpu.SemaphoreType.DMA,
            pltpu.SemaphoreType.DMA,
            pltpu.SemaphoreType.DMA,
        ],
    )(xt, pos1, pos2)

    orows = pl.pallas_call(
        _gmm_body,
        grid_spec=pltpu.PrefetchScalarGridSpec(
            num_scalar_prefetch=1,
            grid=(n_tiles,),
            in_specs=[
                pl.BlockSpec((TM, v), lambda i, te_r: (i, 0)),
                pl.BlockSpec((1, v, out), lambda i, te_r: (te_r[i], 0, 0)),
                pl.BlockSpec((1, 1, out), lambda i, te_r: (te_r[i], 0, 0)),
            ],
            out_specs=pl.BlockSpec((TM, out), lambda i, te_r: (i, 0)),
        ),
        out_shape=jax.ShapeDtypeStruct((rows_pad, out), jnp.float32),
    )(te_flat, xs, We, be.reshape(e, 1, out))

    nchc = TPW // CCHUNK
    p1c = posout[:, 0].reshape(NW, nchc, CCHUNK)
    p2c = posout[:, 1].reshape(NW, nchc, CCHUNK)

    o = pl.kernel(
        _combine_body,
        out_type=jax.ShapeDtypeStruct((t, out), jnp.float32),
        mesh=mesh,
        scratch_types=[
            pltpu.VMEM((nchc, CCHUNK), jnp.int32),
            pltpu.VMEM((nchc, CCHUNK), jnp.int32),
            pltpu.VMEM((2, CCHUNK, out), jnp.float32),
            pltpu.VMEM((2, CCHUNK, out), jnp.float32),
            pltpu.SemaphoreType.DMA,
            pltpu.SemaphoreType.DMA,
        ],
    )(orows, p1c, p2c)

    return o.reshape(n, s, out)
